# baseline (speedup 1.0000x reference)
import functools

import jax
import jax.numpy as jnp
from jax import lax
from jax.experimental import pallas as pl
from jax.experimental.pallas import tpu as pltpu
from jax.experimental.pallas import tpu_sc as plsc

V = 100001
D = 64
B = 4096
L = 50
NC, NS = 2, 16
NW = NC * NS
BPW = B // NW
EPC = 2
NCHUNK = BPW // EPC
ROWS = EPC * L


def _sc_dots(l_cxt, r_cxt, l_lbl, r_lbl, cxt_table, lbl_table):
    mesh = plsc.VectorSubcoreMesh(core_axis_name="c", subcore_axis_name="s")

    @functools.partial(
        pl.kernel,
        out_type=jax.ShapeDtypeStruct((2, NW, BPW), jnp.float32),
        mesh=mesh,
        scratch_types=[
            pltpu.VMEM((2, BPW, L), jnp.int32),
            pltpu.VMEM((2, BPW), jnp.int32),
            pltpu.VMEM((2, BPW, D), jnp.float32),
            pltpu.VMEM((4, L, D), jnp.float32),
            pltpu.VMEM((2, BPW), jnp.float32),
            pltpu.SemaphoreType.DMA,
            pltpu.SemaphoreType.DMA,
            pltpu.SemaphoreType.DMA,
            pltpu.SemaphoreType.DMA,
            pltpu.SemaphoreType.DMA,
        ],
        compiler_params=pltpu.CompilerParams(use_tc_tiling_on_sc=False),
    )
    def kern(l_cxt_hbm, r_cxt_hbm, l_lbl_hbm, r_lbl_hbm,
             cxt_tab_hbm, lbl_tab_hbm, out_hbm,
             idx_v, lidx_v, lrows_v, buf_v, out_v,
             sem0, sem1, sem2, sem3, sem_l):
        wid = lax.axis_index("s") * NC + lax.axis_index("c")
        sems = (sem0, sem1, sem2, sem3)

        pltpu.sync_copy(l_cxt_hbm.at[wid], idx_v.at[0])
        pltpu.sync_copy(r_cxt_hbm.at[wid], idx_v.at[1])
        pltpu.sync_copy(l_lbl_hbm.at[wid], lidx_v.at[0])
        pltpu.sync_copy(r_lbl_hbm.at[wid], lidx_v.at[1])
        pltpu.async_copy(lbl_tab_hbm.at[lidx_v.at[0]], lrows_v.at[0], sem_l).wait()
        pltpu.async_copy(lbl_tab_hbm.at[lidx_v.at[1]], lrows_v.at[1], sem_l).wait()

        lanes = lax.iota(jnp.int32, 16)
        for s in range(2):
            for b in range(4):
                pltpu.async_copy(
                    cxt_tab_hbm.at[idx_v.at[s, b]], buf_v.at[b], sems[b])

            def group16(g, _, s=s):
                dvec = jnp.zeros((16,), jnp.float32)
                for b16 in range(16):
                    bb = 16 * g + b16
                    slot = b16 % 4
                    pltpu.make_async_copy(
                        cxt_tab_hbm.at[idx_v.at[s, slot]], buf_v.at[slot],
                        sems[slot]).wait()
                    lbl = [lrows_v[s, bb, pl.ds(16 * c, 16)]
                           for c in range(4)]

                    def row_acc(l, acc, slot=slot, lbl=lbl):
                        return tuple(
                            acc[c] + buf_v[slot, l, pl.ds(16 * c, 16)]
                            * lbl[c]
                            for c in range(4))

                    z = jnp.zeros((16,), jnp.float32)
                    a = lax.fori_loop(0, L, row_acc, (z, z, z, z),
                                      unroll=10)
                    tot = (a[0] + a[1]) + (a[2] + a[3])
                    for sh in (8, 4, 2, 1):
                        tot = tot + tot.at[lanes ^ sh].get(
                            mode="promise_in_bounds")
                    dvec = jnp.where(lanes == b16, tot, dvec)
                    @pl.when(bb + 4 < BPW)
                    def _(slot=slot, bb=bb, s=s):
                        pltpu.async_copy(
                            cxt_tab_hbm.at[idx_v.at[s, bb + 4]],
                            buf_v.at[slot], sems[slot])
                out_v[s, pl.ds(g * 16, 16)] = dvec
                return 0

            lax.fori_loop(0, BPW // 16, group16, 0)

        pltpu.sync_copy(out_v.at[0], out_hbm.at[0, wid])
        pltpu.sync_copy(out_v.at[1], out_hbm.at[1, wid])

    return kern(l_cxt, r_cxt, l_lbl, r_lbl, cxt_table, lbl_table)


def _tc_loss(dots):
    def body(d_ref, o_ref):
        d = d_ref[...] * (1.0 / L)
        x = jnp.where(jnp.arange(2)[:, None] == 0, d, -d)
        sp = jnp.maximum(x, 0.0) + jnp.log1p(jnp.exp(-jnp.abs(x)))
        o_ref[0, 0] = jnp.sum(sp)

    out = pl.pallas_call(
        body,
        out_shape=jax.ShapeDtypeStruct((1, 1), jnp.float32),
        out_specs=pl.BlockSpec(memory_space=pltpu.SMEM),
    )(dots)
    return out[0, 0]


def kernel(l_cxt, r_cxt, l_lbl, r_lbl, cxt_table, lbl_table):
    dots = _sc_dots(
        l_cxt.astype(jnp.int32).reshape(NW, BPW, L),
        r_cxt.astype(jnp.int32).reshape(NW, BPW, L),
        (l_lbl - V).astype(jnp.int32).reshape(NW, BPW),
        (r_lbl - V).astype(jnp.int32).reshape(NW, BPW),
        cxt_table, lbl_table)
    return _tc_loss(dots.reshape(2, B))


# speedup vs baseline: 1.1200x; 1.1200x over previous
import functools

import jax
import jax.numpy as jnp
from jax import lax
from jax.experimental import pallas as pl
from jax.experimental.pallas import tpu as pltpu
from jax.experimental.pallas import tpu_sc as plsc

V = 100001
D = 64
B = 4096
L = 50
NC, NS = 2, 16
NW = NC * NS
BPW = B // NW
EPC = 2
NCHUNK = BPW // EPC
ROWS = EPC * L


def _sc_dots(l_cxt, r_cxt, l_lbl, r_lbl, cxt_table, lbl_table):
    mesh = plsc.VectorSubcoreMesh(core_axis_name="c", subcore_axis_name="s")

    @functools.partial(
        pl.kernel,
        out_type=jax.ShapeDtypeStruct((2, NW, BPW), jnp.float32),
        mesh=mesh,
        scratch_types=[
            pltpu.VMEM((2, BPW, L), jnp.int32),
            pltpu.VMEM((2, BPW), jnp.int32),
            pltpu.VMEM((2, BPW, D), jnp.float32),
            pltpu.VMEM((8, L, D), jnp.float32),
            pltpu.VMEM((2, BPW), jnp.float32),
            [pltpu.SemaphoreType.DMA] * 8,
            pltpu.SemaphoreType.DMA,
        ],
        compiler_params=pltpu.CompilerParams(use_tc_tiling_on_sc=False),
    )
    def kern(l_cxt_hbm, r_cxt_hbm, l_lbl_hbm, r_lbl_hbm,
             cxt_tab_hbm, lbl_tab_hbm, out_hbm,
             idx_v, lidx_v, lrows_v, buf_v, out_v, sems, sem_l):
        wid = lax.axis_index("s") * NC + lax.axis_index("c")
        base = wid * BPW

        pltpu.sync_copy(l_cxt_hbm.at[pl.ds(base, BPW)], idx_v.at[0])
        pltpu.sync_copy(r_cxt_hbm.at[pl.ds(base, BPW)], idx_v.at[1])
        pltpu.sync_copy(l_lbl_hbm.at[pl.ds(base, BPW)], lidx_v.at[0])
        pltpu.sync_copy(r_lbl_hbm.at[pl.ds(base, BPW)], lidx_v.at[1])
        pltpu.async_copy(lbl_tab_hbm.at[lidx_v.at[0]], lrows_v.at[0], sem_l).wait()
        pltpu.async_copy(lbl_tab_hbm.at[lidx_v.at[1]], lrows_v.at[1], sem_l).wait()

        lanes = lax.iota(jnp.int32, 16)
        for s in range(2):
            for b in range(8):
                pltpu.async_copy(
                    cxt_tab_hbm.at[idx_v.at[s, b]], buf_v.at[b], sems[b])

            def group16(g, _, s=s):
                dvec = jnp.zeros((16,), jnp.float32)
                for b16 in range(16):
                    bb = 16 * g + b16
                    slot = b16 % 8
                    pltpu.make_async_copy(
                        cxt_tab_hbm.at[idx_v.at[s, slot]], buf_v.at[slot],
                        sems[slot]).wait()
                    lbl = [lrows_v[s, bb, pl.ds(16 * c, 16)]
                           for c in range(4)]

                    def row_acc(l, acc, slot=slot, lbl=lbl):
                        return tuple(
                            acc[c] + buf_v[slot, l, pl.ds(16 * c, 16)]
                            * lbl[c]
                            for c in range(4))

                    z = jnp.zeros((16,), jnp.float32)
                    a = lax.fori_loop(0, L, row_acc, (z, z, z, z),
                                      unroll=10)
                    tot = (a[0] + a[1]) + (a[2] + a[3])
                    for sh in (8, 4, 2, 1):
                        tot = tot + tot.at[lanes ^ sh].get(
                            mode="promise_in_bounds")
                    dvec = jnp.where(lanes == b16, tot, dvec)
                    @pl.when(bb + 8 < BPW)
                    def _(slot=slot, bb=bb, s=s):
                        pltpu.async_copy(
                            cxt_tab_hbm.at[idx_v.at[s, bb + 8]],
                            buf_v.at[slot], sems[slot])
                out_v[s, pl.ds(g * 16, 16)] = dvec
                return 0

            lax.fori_loop(0, BPW // 16, group16, 0)

        pltpu.sync_copy(out_v.at[0], out_hbm.at[0, wid])
        pltpu.sync_copy(out_v.at[1], out_hbm.at[1, wid])

    return kern(l_cxt, r_cxt, l_lbl, r_lbl, cxt_table, lbl_table)


def _tc_loss(dots):
    def body(d_ref, o_ref):
        d = d_ref[...] * (1.0 / L)
        x = jnp.where(jnp.arange(2)[:, None] == 0, d, -d)
        sp = jnp.maximum(x, 0.0) + jnp.log1p(jnp.exp(-jnp.abs(x)))
        o_ref[0, 0] = jnp.sum(sp)

    out = pl.pallas_call(
        body,
        out_shape=jax.ShapeDtypeStruct((1, 1), jnp.float32),
        out_specs=pl.BlockSpec(memory_space=pltpu.SMEM),
    )(dots)
    return out[0, 0]


def kernel(l_cxt, r_cxt, l_lbl, r_lbl, cxt_table, lbl_table):
    dots = _sc_dots(
        l_cxt.astype(jnp.int32), r_cxt.astype(jnp.int32),
        (l_lbl - V).astype(jnp.int32), (r_lbl - V).astype(jnp.int32),
        cxt_table, lbl_table)
    return _tc_loss(dots.reshape(2, B))
